# baseline (speedup 1.0000x reference)
import functools

import jax
import jax.numpy as jnp
from jax import lax
from jax.experimental import pallas as pl
from jax.experimental.pallas import tpu as pltpu
from jax.experimental.pallas import tpu_sc as plsc

_N = 10000
_E = 320000
_CHUNK = 50
_NROWS = _E // _CHUNK
_NW = 32
_RPW = _NROWS // _NW
_NSLICE = _N // 16
_ZCH = 125

_mesh = plsc.VectorSubcoreMesh(core_axis_name="c", subcore_axis_name="s",
                               num_cores=2, num_subcores=16)


def _zero_vmem(buf, nrows, width):
    def row(r, _):
        for j in range(width // 16):
            buf[r, pl.ds(j * 16, 16)] = jnp.zeros((16,), jnp.float32)
        return 0
    lax.fori_loop(0, nrows, row, 0)


_NB = 4


def _agg_body(width, row_hbm, col_hbm, table_hbm, out_hbm,
              idx_r, idx_c, rows_v, acc, sg0, sg1, sg2, sg3,
              ss0, ss1, ss2, ss3):
    c = lax.axis_index("c")
    s = lax.axis_index("s")
    sg = (sg0, sg1, sg2, sg3)
    ss = (ss0, ss1, ss2, ss3)

    def zrow(r, _):
        for w in range(width // 16):
            rows_v[0, r, pl.ds(w * 16, 16)] = jnp.zeros((16,), jnp.float32)
        return 0
    lax.fori_loop(0, 25, zrow, 0)
    for k in range(_NSLICE // 25):
        pltpu.async_copy(rows_v.at[0, pl.ds(0, 25), :],
                         acc.at[pl.ds(s * _NSLICE + k * 25, 25), :], sg0)
    for k in range(_NSLICE // 25):
        pltpu.make_async_copy(rows_v.at[0, pl.ds(0, 25), :],
                              acc.at[pl.ds(s * _NSLICE + k * 25, 25), :],
                              sg0).wait()

    plsc.subcore_barrier()

    def g_start(j, b):
        pltpu.async_copy(table_hbm.at[idx_r.at[j]], rows_v.at[b], sg[b])

    def g_wait(j, b):
        pltpu.make_async_copy(table_hbm.at[idx_r.at[j]], rows_v.at[b],
                              sg[b]).wait()

    def s_start(j, b):
        pltpu.async_copy(rows_v.at[b], acc.at[idx_c.at[j]], ss[b], add=True)

    def s_wait(j, b):
        pltpu.make_async_copy(rows_v.at[b], acc.at[idx_c.at[j]],
                              ss[b]).wait()

    for off, hh in ((0, 56), (56, 56), (112, 56), (168, 32)):
        base = (c * 16 + s) * _RPW + off
        pltpu.sync_copy(row_hbm.at[pl.ds(base, hh), :],
                        idx_r.at[pl.ds(0, hh), :])
        pltpu.sync_copy(col_hbm.at[pl.ds(base, hh), :],
                        idx_c.at[pl.ds(0, hh), :])

        g_start(0, 0)
        g_start(1, 1)
        g_wait(0, 0)
        s_start(0, 0)
        g_start(2, 2)
        g_wait(1, 1)
        s_start(1, 1)
        g_start(3, 3)

        @pl.loop(2, hh - 2, step=4)
        def _pipe(j):
            for t in range(4):
                b = (2 + t) % 4
                g_wait(j + t, b)
                s_start(j + t, b)
                s_wait(j + t - 2, (b + 2) % 4)
                g_start(j + t + 2, (b + 2) % 4)

        g_wait(hh - 2, 2)
        s_start(hh - 2, 2)
        s_wait(hh - 4, 0)
        g_wait(hh - 1, 3)
        s_start(hh - 1, 3)
        s_wait(hh - 3, 1)
        s_wait(hh - 2, 2)
        s_wait(hh - 1, 3)

    plsc.subcore_barrier()
    pltpu.sync_copy(acc.at[pl.ds(s * _NSLICE, _NSLICE), :], out_hbm.at[c, s])


def _make_agg(width, interpret=False, tc_tiling=True):
    return pl.kernel(
        functools.partial(_agg_body, width),
        out_type=jax.ShapeDtypeStruct((2, 16, _NSLICE, width), jnp.float32),
        mesh=_mesh,
        interpret=interpret,
        compiler_params=pltpu.CompilerParams(use_tc_tiling_on_sc=tc_tiling),
        scratch_types=[
            pltpu.VMEM((56, _CHUNK), jnp.int32),
            pltpu.VMEM((56, _CHUNK), jnp.int32),
            pltpu.VMEM((_NB, _CHUNK, width), jnp.float32),
            pltpu.VMEM_SHARED((_N, width), jnp.float32),
            pltpu.SemaphoreType.DMA,
            pltpu.SemaphoreType.DMA,
            pltpu.SemaphoreType.DMA,
            pltpu.SemaphoreType.DMA,
            pltpu.SemaphoreType.DMA,
            pltpu.SemaphoreType.DMA,
            pltpu.SemaphoreType.DMA,
            pltpu.SemaphoreType.DMA,
        ],
    )


def _deg_body(col_hbm, out_hbm, idx_c, ones_v, acc, sem):
    c = lax.axis_index("c")
    s = lax.axis_index("s")

    def zrow(r, _):
        ones_v[r, pl.ds(0, 16)] = jnp.zeros((16,), jnp.float32)
        return 0
    lax.fori_loop(0, _CHUNK, zrow, 0)
    for k in range(_NSLICE // 25):
        pltpu.async_copy(ones_v.at[pl.ds(0, 25), :],
                         acc.at[pl.ds(s * _NSLICE + k * 25, 25), :], sem)
    for k in range(_NSLICE // 25):
        pltpu.make_async_copy(ones_v.at[pl.ds(0, 25), :],
                              acc.at[pl.ds(s * _NSLICE + k * 25, 25), :],
                              sem).wait()

    def orow(r, _):
        ones_v[r, pl.ds(0, 16)] = jnp.ones((16,), jnp.float32)
        return 0
    lax.fori_loop(0, _CHUNK, orow, 0)

    base = (c * 16 + s) * _RPW
    pltpu.sync_copy(col_hbm.at[pl.ds(base, _RPW), :], idx_c)
    plsc.subcore_barrier()

    _G = 8

    def group(g, _):
        for t in range(_G):
            pltpu.async_copy(ones_v, acc.at[idx_c.at[g * _G + t]], sem,
                             add=True)
        for t in range(_G):
            pltpu.make_async_copy(ones_v, acc.at[idx_c.at[g * _G + t]],
                                  sem).wait()
        return 0
    lax.fori_loop(0, _RPW // _G, group, 0)

    plsc.subcore_barrier()
    pltpu.sync_copy(acc.at[pl.ds(s * _NSLICE, _NSLICE), :], out_hbm.at[c, s])


def _make_deg(interpret=False):
    return pl.kernel(
        _deg_body,
        out_type=jax.ShapeDtypeStruct((2, 16, _NSLICE, 16), jnp.float32),
        mesh=_mesh,
        interpret=interpret,
        compiler_params=pltpu.CompilerParams(use_tc_tiling_on_sc=False),
        scratch_types=[
            pltpu.VMEM((_RPW, _CHUNK), jnp.int32),
            pltpu.VMEM((_CHUNK, 16), jnp.float32),
            pltpu.VMEM_SHARED((_N, 16), jnp.float32),
            pltpu.SemaphoreType.DMA,
        ],
    )


_deg_kernel = _make_deg()
_agg_kernel = _make_agg(128)


_BR = 1000


def _dis_from_cnt(cnt_blk):
    return lax.rsqrt(cnt_blk[0, :, 0] + cnt_blk[1, :, 0] + 1.0)


def _xw_body(x_ref, w_ref, o_ref):
    o_ref[...] = jnp.dot(x_ref[...], w_ref[...],
                         preferred_element_type=jnp.float32)


def _xw(x, W_enc):
    return pl.pallas_call(
        _xw_body,
        grid=(_N // _BR,),
        in_specs=[pl.BlockSpec((_BR, 128), lambda i: (i, 0)),
                  pl.BlockSpec((128, 128), lambda i: (0, 0))],
        out_specs=pl.BlockSpec((_BR, 128), lambda i: (i, 0)),
        out_shape=jax.ShapeDtypeStruct((_N, 128), jnp.float32),
    )(x, W_enc)


def _xs_body(xw_ref, cnt_ref, o_ref):
    dis = _dis_from_cnt(cnt_ref)
    o_ref[...] = xw_ref[...] * dis[:, None]


def _xs(xw, cnt):
    return pl.pallas_call(
        _xs_body,
        grid=(_N // _BR,),
        in_specs=[pl.BlockSpec((_BR, 128), lambda i: (i, 0)),
                  pl.BlockSpec((2, _BR, 16), lambda i: (0, i, 0))],
        out_specs=pl.BlockSpec((_BR, 128), lambda i: (i, 0)),
        out_shape=jax.ShapeDtypeStruct((_N, 128), jnp.float32),
    )(xw, cnt)


def _hs_body(p_ref, xs_ref, cnt_ref, b_ref, o_ref):
    dis = _dis_from_cnt(cnt_ref)
    t = dis[:, None] * (p_ref[0] + p_ref[1] + xs_ref[...]) + b_ref[...]
    o_ref[...] = jnp.maximum(t, 0.0) * dis[:, None]


def _hs(part1, xs, cnt, b_enc):
    return pl.pallas_call(
        _hs_body,
        grid=(_N // _BR,),
        in_specs=[pl.BlockSpec((2, _BR, 128), lambda i: (0, i, 0)),
                  pl.BlockSpec((_BR, 128), lambda i: (i, 0)),
                  pl.BlockSpec((2, _BR, 16), lambda i: (0, i, 0)),
                  pl.BlockSpec((1, 128), lambda i: (0, 0))],
        out_specs=pl.BlockSpec((_BR, 128), lambda i: (i, 0)),
        out_shape=jax.ShapeDtypeStruct((_N, 128), jnp.float32),
    )(part1, xs, cnt, b_enc)


def _ml_body(p_ref, hs_ref, cnt_ref, w_ref, b_ref, ml_ref, mupad_ref):
    dis = _dis_from_cnt(cnt_ref)
    hagg = dis[:, None] * (p_ref[0] + p_ref[1] + hs_ref[...])
    ml = jnp.dot(hagg, w_ref[...], preferred_element_type=jnp.float32) \
        + b_ref[...]
    ml_ref[...] = ml
    lane = lax.broadcasted_iota(jnp.int32, ml.shape, 1)
    mupad_ref[...] = jnp.where(lane < 64, ml, 0.0)


def _ml(part2, hs, cnt, Wml, bml):
    return pl.pallas_call(
        _ml_body,
        grid=(_N // _BR,),
        in_specs=[pl.BlockSpec((2, _BR, 128), lambda i: (0, i, 0)),
                  pl.BlockSpec((_BR, 128), lambda i: (i, 0)),
                  pl.BlockSpec((2, _BR, 16), lambda i: (0, i, 0)),
                  pl.BlockSpec((128, 128), lambda i: (0, 0)),
                  pl.BlockSpec((1, 128), lambda i: (0, 0))],
        out_specs=[pl.BlockSpec((_BR, 128), lambda i: (i, 0)),
                   pl.BlockSpec((_BR, 128), lambda i: (i, 0))],
        out_shape=[jax.ShapeDtypeStruct((_N, 128), jnp.float32),
                   jax.ShapeDtypeStruct((_N, 128), jnp.float32)],
    )(part2, hs, cnt, Wml, bml)


_BA = 400


def _adj_body(a_ref, b_ref, o_ref):
    o_ref[...] = lax.dot_general(
        a_ref[...], b_ref[...], (((1,), (1,)), ((), ())),
        preferred_element_type=jnp.float32)


def _adj_matmul(mupad):
    return pl.pallas_call(
        _adj_body,
        grid=(_N // _BA,),
        in_specs=[pl.BlockSpec((_BA, 128), lambda i: (i, 0)),
                  pl.BlockSpec((_N, 128), lambda i: (0, 0))],
        out_specs=pl.BlockSpec((_BA, _N), lambda i: (i, 0)),
        out_shape=jax.ShapeDtypeStruct((_N, _N), jnp.float32),
    )(mupad, mupad)


def kernel(x, edge_index, W_enc, b_enc, W1, b1, W2, b2):
    row2d = edge_index[0].reshape(_NROWS, _CHUNK)
    col2d = edge_index[1].reshape(_NROWS, _CHUNK)

    cnt = _deg_kernel(col2d).reshape(2, _N, 16)
    xw = _xw(x, W_enc)
    xs = _xs(xw, cnt)

    part1 = _agg_kernel(row2d, col2d, xs).reshape(2, _N, 128)
    hs = _hs(part1, xs, cnt, b_enc.reshape(1, 128))

    part2 = _agg_kernel(row2d, col2d, hs).reshape(2, _N, 128)
    Wml = jnp.concatenate([W1, W2], axis=1)
    bml = jnp.concatenate([b1, b2], axis=0).reshape(1, 128)
    ml, mupad = _ml(part2, hs, cnt, Wml, bml)

    Adj = _adj_matmul(mupad)
    mu = ml[:, :64]
    logvar = ml[:, 64:]
    return (Adj, mu, logvar)


# speedup vs baseline: 1.0350x; 1.0350x over previous
import functools

import jax
import jax.numpy as jnp
from jax import lax
from jax.experimental import pallas as pl
from jax.experimental.pallas import tpu as pltpu
from jax.experimental.pallas import tpu_sc as plsc

_N = 10000
_E = 320000
_CHUNK = 125
_NROWS = _E // _CHUNK
_NW = 32
_RPW = _NROWS // _NW
_NSLICE = _N // 16

_mesh = plsc.VectorSubcoreMesh(core_axis_name="c", subcore_axis_name="s",
                               num_cores=2, num_subcores=16)


def _zero_vmem(buf, nrows, width):
    def row(r, _):
        for j in range(width // 16):
            buf[r, pl.ds(j * 16, 16)] = jnp.zeros((16,), jnp.float32)
        return 0
    lax.fori_loop(0, nrows, row, 0)


_H = _RPW // 2


def _agg_body(width, row_hbm, col_hbm, table_hbm, out_hbm,
              idx_r, idx_c, rows_v, acc, sg0, sg1, ss0, ss1):
    c = lax.axis_index("c")
    s = lax.axis_index("s")
    sg = (sg0, sg1)
    ss = (ss0, ss1)

    def zrow(r, _):
        for w in range(width // 16):
            rows_v[0, r, pl.ds(w * 16, 16)] = jnp.zeros((16,), jnp.float32)
        return 0
    lax.fori_loop(0, _CHUNK, zrow, 0)
    for k in range(_NSLICE // _CHUNK):
        pltpu.sync_copy(rows_v.at[0],
                        acc.at[pl.ds(s * _NSLICE + k * _CHUNK, _CHUNK), :])
    plsc.subcore_barrier()

    def g_start(j, b):
        pltpu.async_copy(table_hbm.at[idx_r.at[j]], rows_v.at[b], sg[b])

    def g_wait(j, b):
        pltpu.make_async_copy(table_hbm.at[idx_r.at[j]], rows_v.at[b],
                              sg[b]).wait()

    def s_start(j, b):
        pltpu.async_copy(rows_v.at[b], acc.at[idx_c.at[j]], ss[b], add=True)

    def s_wait(j, b):
        pltpu.make_async_copy(rows_v.at[b], acc.at[idx_c.at[j]],
                              ss[b]).wait()

    for h in range(2):
        base = (c * 16 + s) * _RPW + h * _H
        pltpu.sync_copy(row_hbm.at[pl.ds(base, _H), :], idx_r)
        pltpu.sync_copy(col_hbm.at[pl.ds(base, _H), :], idx_c)

        g_start(0, 0)
        g_wait(0, 0)
        s_start(0, 0)
        g_start(1, 1)

        @pl.loop(1, _H - 1, step=2)
        def _pipe(j):
            g_wait(j, 1)
            s_start(j, 1)
            s_wait(j - 1, 0)
            g_start(j + 1, 0)
            g_wait(j + 1, 0)
            s_start(j + 1, 0)
            s_wait(j, 1)
            g_start(j + 2, 1)

        g_wait(_H - 1, 1)
        s_start(_H - 1, 1)
        s_wait(_H - 2, 0)
        s_wait(_H - 1, 1)

    plsc.subcore_barrier()
    pltpu.sync_copy(acc.at[pl.ds(s * _NSLICE, _NSLICE), :], out_hbm.at[c, s])


def _make_agg(width, interpret=False, tc_tiling=True):
    return pl.kernel(
        functools.partial(_agg_body, width),
        out_type=jax.ShapeDtypeStruct((2, 16, _NSLICE, width), jnp.float32),
        mesh=_mesh,
        interpret=interpret,
        compiler_params=pltpu.CompilerParams(use_tc_tiling_on_sc=tc_tiling),
        scratch_types=[
            pltpu.VMEM((_H, _CHUNK), jnp.int32),
            pltpu.VMEM((_H, _CHUNK), jnp.int32),
            pltpu.VMEM((2, _CHUNK, width), jnp.float32),
            pltpu.VMEM_SHARED((_N, width), jnp.float32),
            pltpu.SemaphoreType.DMA,
            pltpu.SemaphoreType.DMA,
            pltpu.SemaphoreType.DMA,
            pltpu.SemaphoreType.DMA,
        ],
    )


def _deg_body(col_hbm, out_hbm, idx_c, ones_v, acc, sem):
    c = lax.axis_index("c")
    s = lax.axis_index("s")

    def zrow(r, _):
        ones_v[r, pl.ds(0, 16)] = jnp.zeros((16,), jnp.float32)
        return 0
    lax.fori_loop(0, _CHUNK, zrow, 0)
    for k in range(_NSLICE // 25):
        pltpu.async_copy(ones_v.at[pl.ds(0, 25), :],
                         acc.at[pl.ds(s * _NSLICE + k * 25, 25), :], sem)
    for k in range(_NSLICE // 25):
        pltpu.make_async_copy(ones_v.at[pl.ds(0, 25), :],
                              acc.at[pl.ds(s * _NSLICE + k * 25, 25), :],
                              sem).wait()

    def orow(r, _):
        ones_v[r, pl.ds(0, 16)] = jnp.ones((16,), jnp.float32)
        return 0
    lax.fori_loop(0, _CHUNK, orow, 0)

    base = (c * 16 + s) * _RPW
    pltpu.sync_copy(col_hbm.at[pl.ds(base, _RPW), :], idx_c)
    plsc.subcore_barrier()

    _G = 8

    def group(g, _):
        for t in range(_G):
            pltpu.async_copy(ones_v, acc.at[idx_c.at[g * _G + t]], sem,
                             add=True)
        for t in range(_G):
            pltpu.make_async_copy(ones_v, acc.at[idx_c.at[g * _G + t]],
                                  sem).wait()
        return 0
    lax.fori_loop(0, _RPW // _G, group, 0)

    plsc.subcore_barrier()
    pltpu.sync_copy(acc.at[pl.ds(s * _NSLICE, _NSLICE), :], out_hbm.at[c, s])


def _make_deg(interpret=False):
    return pl.kernel(
        _deg_body,
        out_type=jax.ShapeDtypeStruct((2, 16, _NSLICE, 16), jnp.float32),
        mesh=_mesh,
        interpret=interpret,
        compiler_params=pltpu.CompilerParams(use_tc_tiling_on_sc=False),
        scratch_types=[
            pltpu.VMEM((_RPW, _CHUNK), jnp.int32),
            pltpu.VMEM((_CHUNK, 16), jnp.float32),
            pltpu.VMEM_SHARED((_N, 16), jnp.float32),
            pltpu.SemaphoreType.DMA,
        ],
    )


_deg_kernel = _make_deg()
_agg_kernel = _make_agg(128)


_BR = 1000


def _dis_from_cnt(cnt_blk):
    return lax.rsqrt(cnt_blk[0, :, 0] + cnt_blk[1, :, 0] + 1.0)


def _xs_body(x_ref, w_ref, cnt_ref, o_ref):
    dis = _dis_from_cnt(cnt_ref)
    xw = jnp.dot(x_ref[...], w_ref[...], preferred_element_type=jnp.float32)
    o_ref[...] = xw * dis[:, None]


def _xs(x, W_enc, cnt):
    return pl.pallas_call(
        _xs_body,
        grid=(_N // _BR,),
        in_specs=[pl.BlockSpec((_BR, 128), lambda i: (i, 0)),
                  pl.BlockSpec((128, 128), lambda i: (0, 0)),
                  pl.BlockSpec((2, _BR, 16), lambda i: (0, i, 0))],
        out_specs=pl.BlockSpec((_BR, 128), lambda i: (i, 0)),
        out_shape=jax.ShapeDtypeStruct((_N, 128), jnp.float32),
    )(x, W_enc, cnt)


def _hs_body(p_ref, xs_ref, cnt_ref, b_ref, o_ref):
    dis = _dis_from_cnt(cnt_ref)
    t = dis[:, None] * (p_ref[0] + p_ref[1] + xs_ref[...]) + b_ref[...]
    o_ref[...] = jnp.maximum(t, 0.0) * dis[:, None]


def _hs(part1, xs, cnt, b_enc):
    return pl.pallas_call(
        _hs_body,
        grid=(_N // _BR,),
        in_specs=[pl.BlockSpec((2, _BR, 128), lambda i: (0, i, 0)),
                  pl.BlockSpec((_BR, 128), lambda i: (i, 0)),
                  pl.BlockSpec((2, _BR, 16), lambda i: (0, i, 0)),
                  pl.BlockSpec((1, 128), lambda i: (0, 0))],
        out_specs=pl.BlockSpec((_BR, 128), lambda i: (i, 0)),
        out_shape=jax.ShapeDtypeStruct((_N, 128), jnp.float32),
    )(part1, xs, cnt, b_enc)


def _ml_body(p_ref, hs_ref, cnt_ref, w_ref, b_ref, ml_ref, mupad_ref):
    dis = _dis_from_cnt(cnt_ref)
    hagg = dis[:, None] * (p_ref[0] + p_ref[1] + hs_ref[...])
    ml = jnp.dot(hagg, w_ref[...], preferred_element_type=jnp.float32) \
        + b_ref[...]
    ml_ref[...] = ml
    lane = lax.broadcasted_iota(jnp.int32, ml.shape, 1)
    mupad_ref[...] = jnp.where(lane < 64, ml, 0.0)


def _ml(part2, hs, cnt, Wml, bml):
    return pl.pallas_call(
        _ml_body,
        grid=(_N // _BR,),
        in_specs=[pl.BlockSpec((2, _BR, 128), lambda i: (0, i, 0)),
                  pl.BlockSpec((_BR, 128), lambda i: (i, 0)),
                  pl.BlockSpec((2, _BR, 16), lambda i: (0, i, 0)),
                  pl.BlockSpec((128, 128), lambda i: (0, 0)),
                  pl.BlockSpec((1, 128), lambda i: (0, 0))],
        out_specs=[pl.BlockSpec((_BR, 128), lambda i: (i, 0)),
                   pl.BlockSpec((_BR, 128), lambda i: (i, 0))],
        out_shape=[jax.ShapeDtypeStruct((_N, 128), jnp.float32),
                   jax.ShapeDtypeStruct((_N, 128), jnp.float32)],
    )(part2, hs, cnt, Wml, bml)


_BA = 400


def _adj_body(a_ref, b_ref, o_ref):
    o_ref[...] = lax.dot_general(
        a_ref[...], b_ref[...], (((1,), (1,)), ((), ())),
        preferred_element_type=jnp.float32)


def _adj_matmul(mupad):
    return pl.pallas_call(
        _adj_body,
        grid=(_N // _BA,),
        in_specs=[pl.BlockSpec((_BA, 128), lambda i: (i, 0)),
                  pl.BlockSpec((_N, 128), lambda i: (0, 0))],
        out_specs=pl.BlockSpec((_BA, _N), lambda i: (i, 0)),
        out_shape=jax.ShapeDtypeStruct((_N, _N), jnp.float32),
    )(mupad, mupad)


def kernel(x, edge_index, W_enc, b_enc, W1, b1, W2, b2):
    row2d = edge_index[0].reshape(_NROWS, _CHUNK)
    col2d = edge_index[1].reshape(_NROWS, _CHUNK)

    cnt = _deg_kernel(col2d).reshape(2, _N, 16)
    xs = _xs(x, W_enc, cnt)

    part1 = _agg_kernel(row2d, col2d, xs).reshape(2, _N, 128)
    hs = _hs(part1, xs, cnt, b_enc.reshape(1, 128))

    part2 = _agg_kernel(row2d, col2d, hs).reshape(2, _N, 128)
    Wml = jnp.concatenate([W1, W2], axis=1)
    bml = jnp.concatenate([b1, b2], axis=0).reshape(1, 128)
    ml, mupad = _ml(part2, hs, cnt, Wml, bml)

    Adj = _adj_matmul(mupad)
    mu = ml[:, :64]
    logvar = ml[:, 64:]
    return (Adj, mu, logvar)


# speedup vs baseline: 1.0477x; 1.0122x over previous
import functools

import jax
import jax.numpy as jnp
from jax import lax
from jax.experimental import pallas as pl
from jax.experimental.pallas import tpu as pltpu
from jax.experimental.pallas import tpu_sc as plsc

_N = 10000
_E = 320000
_CHUNK = 125
_NROWS = _E // _CHUNK
_NW = 32
_RPW = _NROWS // _NW
_NSLICE = _N // 16

_mesh = plsc.VectorSubcoreMesh(core_axis_name="c", subcore_axis_name="s",
                               num_cores=2, num_subcores=16)


def _zero_vmem(buf, nrows, width):
    def row(r, _):
        for j in range(width // 16):
            buf[r, pl.ds(j * 16, 16)] = jnp.zeros((16,), jnp.float32)
        return 0
    lax.fori_loop(0, nrows, row, 0)


_H = _RPW // 2


def _agg_body(width, row_hbm, col_hbm, table_hbm, out_hbm,
              idx_r, idx_c, rows_v, acc, sg0, sg1, ss0, ss1):
    c = lax.axis_index("c")
    s = lax.axis_index("s")
    sg = (sg0, sg1)
    ss = (ss0, ss1)

    def zrow(r, _):
        for w in range(width // 16):
            rows_v[0, r, pl.ds(w * 16, 16)] = jnp.zeros((16,), jnp.float32)
        return 0
    lax.fori_loop(0, _CHUNK, zrow, 0)
    for k in range(_NSLICE // _CHUNK):
        pltpu.sync_copy(rows_v.at[0],
                        acc.at[pl.ds(s * _NSLICE + k * _CHUNK, _CHUNK), :])
    plsc.subcore_barrier()

    def g_start(j, b):
        pltpu.async_copy(table_hbm.at[idx_r.at[j]], rows_v.at[b], sg[b])

    def g_wait(j, b):
        pltpu.make_async_copy(table_hbm.at[idx_r.at[j]], rows_v.at[b],
                              sg[b]).wait()

    def s_start(j, b):
        pltpu.async_copy(rows_v.at[b], acc.at[idx_c.at[j]], ss[b], add=True)

    def s_wait(j, b):
        pltpu.make_async_copy(rows_v.at[b], acc.at[idx_c.at[j]],
                              ss[b]).wait()

    for h in range(2):
        base = (c * 16 + s) * _RPW + h * _H
        pltpu.sync_copy(row_hbm.at[pl.ds(base, _H), :], idx_r)
        pltpu.sync_copy(col_hbm.at[pl.ds(base, _H), :], idx_c)

        g_start(0, 0)
        g_wait(0, 0)
        s_start(0, 0)
        g_start(1, 1)

        @pl.loop(1, _H - 1, step=2)
        def _pipe(j):
            g_wait(j, 1)
            s_start(j, 1)
            s_wait(j - 1, 0)
            g_start(j + 1, 0)
            g_wait(j + 1, 0)
            s_start(j + 1, 0)
            s_wait(j, 1)
            g_start(j + 2, 1)

        g_wait(_H - 1, 1)
        s_start(_H - 1, 1)
        s_wait(_H - 2, 0)
        s_wait(_H - 1, 1)

    plsc.subcore_barrier()
    pltpu.sync_copy(acc.at[pl.ds(s * _NSLICE, _NSLICE), :], out_hbm.at[c, s])


def _make_agg(width, interpret=False, tc_tiling=True):
    return pl.kernel(
        functools.partial(_agg_body, width),
        out_type=jax.ShapeDtypeStruct((2, 16, _NSLICE, width), jnp.float32),
        mesh=_mesh,
        interpret=interpret,
        compiler_params=pltpu.CompilerParams(use_tc_tiling_on_sc=tc_tiling),
        scratch_types=[
            pltpu.VMEM((_H, _CHUNK), jnp.int32),
            pltpu.VMEM((_H, _CHUNK), jnp.int32),
            pltpu.VMEM((2, _CHUNK, width), jnp.float32),
            pltpu.VMEM_SHARED((_N, width), jnp.float32),
            pltpu.SemaphoreType.DMA,
            pltpu.SemaphoreType.DMA,
            pltpu.SemaphoreType.DMA,
            pltpu.SemaphoreType.DMA,
        ],
    )


def _deg_body(col_hbm, out_hbm, idx_c, ones_v, acc, sem):
    c = lax.axis_index("c")
    s = lax.axis_index("s")

    def zrow(r, _):
        ones_v[r, pl.ds(0, 16)] = jnp.zeros((16,), jnp.float32)
        return 0
    lax.fori_loop(0, _CHUNK, zrow, 0)
    for k in range(_NSLICE // 25):
        pltpu.async_copy(ones_v.at[pl.ds(0, 25), :],
                         acc.at[pl.ds(s * _NSLICE + k * 25, 25), :], sem)
    for k in range(_NSLICE // 25):
        pltpu.make_async_copy(ones_v.at[pl.ds(0, 25), :],
                              acc.at[pl.ds(s * _NSLICE + k * 25, 25), :],
                              sem).wait()

    def orow(r, _):
        ones_v[r, pl.ds(0, 16)] = jnp.ones((16,), jnp.float32)
        return 0
    lax.fori_loop(0, _CHUNK, orow, 0)

    base = (c * 16 + s) * _RPW
    pltpu.sync_copy(col_hbm.at[pl.ds(base, _RPW), :], idx_c)
    plsc.subcore_barrier()

    _G = 8

    def group(g, _):
        for t in range(_G):
            pltpu.async_copy(ones_v, acc.at[idx_c.at[g * _G + t]], sem,
                             add=True)
        for t in range(_G):
            pltpu.make_async_copy(ones_v, acc.at[idx_c.at[g * _G + t]],
                                  sem).wait()
        return 0
    lax.fori_loop(0, _RPW // _G, group, 0)

    plsc.subcore_barrier()
    pltpu.sync_copy(acc.at[pl.ds(s * _NSLICE, _NSLICE), :], out_hbm.at[c, s])


def _make_deg(interpret=False):
    return pl.kernel(
        _deg_body,
        out_type=jax.ShapeDtypeStruct((2, 16, _NSLICE, 16), jnp.float32),
        mesh=_mesh,
        interpret=interpret,
        compiler_params=pltpu.CompilerParams(use_tc_tiling_on_sc=False),
        scratch_types=[
            pltpu.VMEM((_RPW, _CHUNK), jnp.int32),
            pltpu.VMEM((_CHUNK, 16), jnp.float32),
            pltpu.VMEM_SHARED((_N, 16), jnp.float32),
            pltpu.SemaphoreType.DMA,
        ],
    )


_deg_kernel = _make_deg()
_agg_kernel = _make_agg(128)


_BR = 1000


def _dis_from_cnt(cnt_blk):
    return lax.rsqrt(cnt_blk[0, :, 0] + cnt_blk[1, :, 0] + 1.0)


def _xs_body(x_ref, w_ref, cnt_ref, o_ref):
    dis = _dis_from_cnt(cnt_ref)
    xw = jnp.dot(x_ref[...], w_ref[...], preferred_element_type=jnp.float32)
    o_ref[...] = xw * dis[:, None]


def _xs(x, W_enc, cnt):
    return pl.pallas_call(
        _xs_body,
        grid=(_N // _BR,),
        in_specs=[pl.BlockSpec((_BR, 128), lambda i: (i, 0)),
                  pl.BlockSpec((128, 128), lambda i: (0, 0)),
                  pl.BlockSpec((2, _BR, 16), lambda i: (0, i, 0))],
        out_specs=pl.BlockSpec((_BR, 128), lambda i: (i, 0)),
        out_shape=jax.ShapeDtypeStruct((_N, 128), jnp.float32),
    )(x, W_enc, cnt)


def _hs_body(p_ref, xs_ref, cnt_ref, b_ref, o_ref):
    dis = _dis_from_cnt(cnt_ref)
    t = dis[:, None] * (p_ref[0] + p_ref[1] + xs_ref[...]) + b_ref[...]
    o_ref[...] = jnp.maximum(t, 0.0) * dis[:, None]


def _hs(part1, xs, cnt, b_enc):
    return pl.pallas_call(
        _hs_body,
        grid=(_N // _BR,),
        in_specs=[pl.BlockSpec((2, _BR, 128), lambda i: (0, i, 0)),
                  pl.BlockSpec((_BR, 128), lambda i: (i, 0)),
                  pl.BlockSpec((2, _BR, 16), lambda i: (0, i, 0)),
                  pl.BlockSpec((1, 128), lambda i: (0, 0))],
        out_specs=pl.BlockSpec((_BR, 128), lambda i: (i, 0)),
        out_shape=jax.ShapeDtypeStruct((_N, 128), jnp.float32),
    )(part1, xs, cnt, b_enc)


def _ml_body(p_ref, hs_ref, cnt_ref, w_ref, b_ref,
             mu_ref, lv_ref, mupad_ref):
    dis = _dis_from_cnt(cnt_ref)
    hagg = dis[:, None] * (p_ref[0] + p_ref[1] + hs_ref[...])
    ml = jnp.dot(hagg, w_ref[...], preferred_element_type=jnp.float32) \
        + b_ref[...]
    mu_ref[...] = ml[:, :64]
    lv_ref[...] = ml[:, 64:]
    lane = lax.broadcasted_iota(jnp.int32, ml.shape, 1)
    mupad_ref[...] = jnp.where(lane < 64, ml, 0.0)


def _ml(part2, hs, cnt, Wml, bml):
    return pl.pallas_call(
        _ml_body,
        grid=(_N // _BR,),
        in_specs=[pl.BlockSpec((2, _BR, 128), lambda i: (0, i, 0)),
                  pl.BlockSpec((_BR, 128), lambda i: (i, 0)),
                  pl.BlockSpec((2, _BR, 16), lambda i: (0, i, 0)),
                  pl.BlockSpec((128, 128), lambda i: (0, 0)),
                  pl.BlockSpec((1, 128), lambda i: (0, 0))],
        out_specs=[pl.BlockSpec((_BR, 64), lambda i: (i, 0)),
                   pl.BlockSpec((_BR, 64), lambda i: (i, 0)),
                   pl.BlockSpec((_BR, 128), lambda i: (i, 0))],
        out_shape=[jax.ShapeDtypeStruct((_N, 64), jnp.float32),
                   jax.ShapeDtypeStruct((_N, 64), jnp.float32),
                   jax.ShapeDtypeStruct((_N, 128), jnp.float32)],
    )(part2, hs, cnt, Wml, bml)


_BA = 400


def _adj_body(a_ref, b_ref, o_ref):
    o_ref[...] = lax.dot_general(
        a_ref[...], b_ref[...], (((1,), (1,)), ((), ())),
        preferred_element_type=jnp.float32)


def _adj_matmul(mupad):
    return pl.pallas_call(
        _adj_body,
        grid=(_N // _BA,),
        in_specs=[pl.BlockSpec((_BA, 128), lambda i: (i, 0)),
                  pl.BlockSpec((_N, 128), lambda i: (0, 0))],
        out_specs=pl.BlockSpec((_BA, _N), lambda i: (i, 0)),
        out_shape=jax.ShapeDtypeStruct((_N, _N), jnp.float32),
    )(mupad, mupad)


def kernel(x, edge_index, W_enc, b_enc, W1, b1, W2, b2):
    row2d = edge_index[0].reshape(_NROWS, _CHUNK)
    col2d = edge_index[1].reshape(_NROWS, _CHUNK)

    cnt = _deg_kernel(col2d).reshape(2, _N, 16)
    xs = _xs(x, W_enc, cnt)

    part1 = _agg_kernel(row2d, col2d, xs).reshape(2, _N, 128)
    hs = _hs(part1, xs, cnt, b_enc.reshape(1, 128))

    part2 = _agg_kernel(row2d, col2d, hs).reshape(2, _N, 128)
    Wml = jnp.concatenate([W1, W2], axis=1)
    bml = jnp.concatenate([b1, b2], axis=0).reshape(1, 128)
    mu, logvar, mupad = _ml(part2, hs, cnt, Wml, bml)

    Adj = _adj_matmul(mupad)
    return (Adj, mu, logvar)
